# baseline (speedup 1.0000x reference)
import functools

import jax
import jax.numpy as jnp
from jax import lax
from jax.experimental import pallas as pl
from jax.experimental.pallas import tpu as pltpu
from jax.experimental.pallas import tpu_sc as plsc

_B = 16384
_D = 128
_NC = 8
_GAMMA2 = 0.25
_CENTER_MARGIN = 1.0
_LAMBDA_CENTER = 0.001
_EPS = 1e-06

_CH = 2048
_NROW = _B // _CH

_SC_S = 4096
_TC_S = _B - _SC_S
_TC_NCH = _TC_S // _CH
_SC_NROW = _SC_S // _CH
_NBUF = 4

_SC_NCORES = 2
_SC_NSUB = 16
_SC_W = _SC_NCORES * _SC_NSUB
_SC_R = _SC_S // _SC_W
_SC_G = _SC_R // 16


def _tc_produce_body(centers_ref, x_hbm, d2_out, xbuf, sem):
    c = centers_ref[...]
    cn = jnp.sum(c * c, axis=1, keepdims=True)
    ones = jnp.ones((1, _D), jnp.float32)

    def copy(i):
        return pltpu.make_async_copy(
            x_hbm.at[pl.ds(i * _CH, _CH), :], xbuf.at[i % _NBUF],
            sem.at[i % _NBUF])

    for i in range(_NBUF):
        copy(i).start()

    for i in range(_TC_NCH):
        copy(i).wait()
        x = xbuf[i % _NBUF]
        if i + _NBUF < _TC_NCH:
            copy(i + _NBUF).start()
        cxT = lax.dot_general(c, x, (((1,), (1,)), ((), ())),
                              preferred_element_type=jnp.float32)
        xnT = lax.dot_general(ones, x * x, (((1,), (1,)), ((), ())),
                              preferred_element_type=jnp.float32)
        g = jnp.min(cn - 2.0 * cxT, axis=0, keepdims=True)
        d2_out[pl.ds(i, 1), :] = jnp.maximum(g + xnT, 0.0)


def _sc_produce_body(x_hbm, c_hbm, out_hbm, xbuf, cbuf, obuf):
    wid = lax.axis_index("s") * _SC_NCORES + lax.axis_index("c")
    base = wid * _SC_R
    pltpu.sync_copy(x_hbm.at[pl.ds(_TC_S + base, _SC_R), :], xbuf)
    pltpu.sync_copy(c_hbm, cbuf)
    lane = lax.iota(jnp.int32, 16)
    for g in range(_SC_G):
        rows = g * 16 + lane

        def step(j, accs):
            new = list(accs)
            for k in range(8):
                d = j * 8 + k
                col = jnp.full((16,), d, jnp.int32)
                xv = plsc.load_gather(xbuf, [rows, col])
                for cc in range(_NC):
                    cbc = plsc.load_gather(
                        cbuf, [jnp.full((16,), cc, jnp.int32), col])
                    t = xv - cbc
                    new[cc] = new[cc] + t * t
            return tuple(new)

        accs = lax.fori_loop(
            0, _D // 8, step, tuple(jnp.zeros((16,), jnp.float32)
                                    for _ in range(_NC)))
        m = accs[0]
        for cc in range(1, _NC):
            m = jnp.minimum(m, accs[cc])
        obuf[pl.ds(g * 16, 16)] = m
    pltpu.sync_copy(obuf, out_hbm.at[pl.ds(base, _SC_R)])


def _select_body(labels_ref, centers_ref, d2m_ref, d2t_ref, out_ref, d2s):
    d2s[pl.ds(0, _TC_NCH), :] = d2m_ref[...]
    d2s[pl.ds(_TC_NCH, _SC_NROW), :] = jnp.maximum(d2t_ref[...], 0.0)

    c = centers_ref[...]
    lab = labels_ref[...]
    real = lab == 0
    num_real_f = jnp.sum(jnp.where(real, 1.0, 0.0))
    num_real = num_real_f.astype(jnp.int32)
    num_forged = _B - num_real
    k_real = jnp.maximum(1, (7 * num_real + 9) // 10)
    k_forged = jnp.maximum(1, (7 * num_forged + 9) // 10)
    k_real_f = k_real.astype(jnp.float32)
    k_forged_f = k_forged.astype(jnp.float32)

    d2a = d2s[...]
    bits = lax.bitcast_convert_type(d2a, jnp.int32)
    rbits = jnp.where(real, bits, jnp.int32(-1))
    fbits = jnp.where(real, jnp.int32(0x7FFFFFFF), bits)

    bmin = lax.bitcast_convert_type(jnp.min(d2a), jnp.int32)
    bmax = lax.bitcast_convert_type(jnp.max(d2a), jnp.int32) + 1

    def it(_, carry):
        lo_r, hi_r, lo_f, hi_f = carry
        mid_r = lo_r + (hi_r - lo_r) // 2
        mid_f = lo_f + (hi_f - lo_f) // 2
        contrib = ((rbits >= mid_r).astype(jnp.int32)
                   + ((fbits < mid_f).astype(jnp.int32) << 15))
        s = jnp.sum(contrib)
        cnt_r = s & 32767
        cnt_f = s >> 15
        ge = cnt_r >= k_real
        lo_r = jnp.where(ge, mid_r, lo_r)
        hi_r = jnp.where(ge, hi_r, mid_r)
        lt = cnt_f < k_forged
        lo_f = jnp.where(lt, mid_f, lo_f)
        hi_f = jnp.where(lt, hi_f, mid_f)
        return lo_r, hi_r, lo_f, hi_f

    lo_r, _, lo_f, _ = lax.fori_loop(0, 20, it, (bmin, bmax, bmin, bmax))

    v_r = lax.bitcast_convert_type(lo_r, jnp.float32)
    gt = rbits > lo_r
    sum_gt = jnp.sum(jnp.where(gt, d2a, 0.0))
    cnt_gt = jnp.sum(jnp.where(gt, 1.0, 0.0))
    top_sum = sum_gt + (k_real_f - cnt_gt) * v_r
    real_loss = top_sum / (2.0 * (k_real_f + _EPS))
    real_loss = jnp.where(num_real > 0, real_loss, 0.0)

    v_f = lax.bitcast_convert_type(lo_f, jnp.float32)
    ltm = fbits < lo_f
    sum_lt = jnp.sum(jnp.where(ltm, d2a, 0.0))
    cnt_lt = jnp.sum(jnp.where(ltm, 1.0, 0.0))
    bot_sum = sum_lt + (k_forged_f - cnt_lt) * v_f
    avg_forged = bot_sum / (2.0 * (k_forged_f + _EPS))
    forged_term = jnp.where(num_forged > 0,
                            jnp.minimum(avg_forged, _GAMMA2), 0.0)

    cc = lax.dot_general(c, c, (((1,), (1,)), ((), ())),
                         preferred_element_type=jnp.float32)
    cn2 = jnp.sum(c * c, axis=1)
    d2m = jnp.maximum(cn2[:, None] + cn2[None, :] - 2.0 * cc, 0.0)
    ii = lax.broadcasted_iota(jnp.int32, (_NC, _NC), 0)
    jj = lax.broadcasted_iota(jnp.int32, (_NC, _NC), 1)
    upper = jj > ii
    dist = jnp.sqrt(d2m + _EPS)
    hinge = jnp.maximum(_CENTER_MARGIN - dist, 0.0)
    num_pairs = _NC * (_NC - 1) // 2
    repulsion = _LAMBDA_CENTER * (
        jnp.sum(jnp.where(upper, hinge, 0.0)) / (num_pairs + _EPS))

    out_ref[0, 0] = real_loss - forged_term + repulsion


def kernel(cls_global, labels, centers):
    d2_main = pl.pallas_call(
        _tc_produce_body,
        in_specs=[
            pl.BlockSpec(memory_space=pltpu.VMEM),
            pl.BlockSpec(memory_space=pl.ANY),
        ],
        out_specs=pl.BlockSpec(memory_space=pltpu.VMEM),
        out_shape=jax.ShapeDtypeStruct((_TC_NCH, _CH), jnp.float32),
        scratch_shapes=[
            pltpu.VMEM((_NBUF, _CH, _D), jnp.float32),
            pltpu.SemaphoreType.DMA((_NBUF,)),
        ],
    )(centers, cls_global)

    d2_tail = pl.kernel(
        _sc_produce_body,
        out_type=jax.ShapeDtypeStruct((_SC_S,), jnp.float32),
        mesh=plsc.VectorSubcoreMesh(core_axis_name="c",
                                    subcore_axis_name="s"),
        compiler_params=pltpu.CompilerParams(needs_layout_passes=False),
        scratch_types=[
            pltpu.VMEM((_SC_R, _D), jnp.float32),
            pltpu.VMEM((_NC, _D), jnp.float32),
            pltpu.VMEM((_SC_R,), jnp.float32),
        ],
    )(cls_global, centers)

    labels2d = labels.reshape(_NROW, _CH)
    out = pl.pallas_call(
        _select_body,
        in_specs=[
            pl.BlockSpec(memory_space=pltpu.VMEM),
            pl.BlockSpec(memory_space=pltpu.VMEM),
            pl.BlockSpec(memory_space=pltpu.VMEM),
            pl.BlockSpec(memory_space=pltpu.VMEM),
        ],
        out_specs=pl.BlockSpec(memory_space=pltpu.SMEM),
        out_shape=jax.ShapeDtypeStruct((1, 1), jnp.float32),
        scratch_shapes=[pltpu.VMEM((_NROW, _CH), jnp.float32)],
    )(labels2d, centers, d2_main, d2_tail.reshape(_SC_NROW, _CH))
    return out[0, 0]


# speedup vs baseline: 4.8201x; 4.8201x over previous
import jax
import jax.numpy as jnp
from jax import lax
from jax.experimental import pallas as pl
from jax.experimental.pallas import tpu as pltpu

_B = 16384
_D = 128
_NC = 8
_GAMMA2 = 0.25
_CENTER_MARGIN = 1.0
_LAMBDA_CENTER = 0.001
_EPS = 1e-06

_CH = 2048
_NCH = _B // _CH
_NBUF = 4


def _body(labels_ref, centers_ref, x_hbm, out_ref, xbuf, d2_ref, sem):
    c = centers_ref[...]
    cn = jnp.sum(c * c, axis=1, keepdims=True)
    ones = jnp.ones((1, _D), jnp.float32)

    def copy(i):
        return pltpu.make_async_copy(
            x_hbm.at[pl.ds(i * _CH, _CH), :], xbuf.at[i % _NBUF],
            sem.at[i % _NBUF])

    for i in range(_NBUF):
        copy(i).start()

    lab = labels_ref[...]
    real = lab == 0
    num_real_f = jnp.sum(jnp.where(real, 1.0, 0.0))
    num_real = num_real_f.astype(jnp.int32)
    num_forged = _B - num_real
    k_real = jnp.maximum(1, (7 * num_real + 9) // 10)
    k_forged = jnp.maximum(1, (7 * num_forged + 9) // 10)
    k_real_f = k_real.astype(jnp.float32)
    k_forged_f = k_forged.astype(jnp.float32)

    for i in range(_NCH):
        copy(i).wait()
        x = xbuf[i % _NBUF]
        if i + _NBUF < _NCH:
            copy(i + _NBUF).start()
        cxT = lax.dot_general(c, x, (((1,), (1,)), ((), ())),
                              preferred_element_type=jnp.float32)
        xnT = lax.dot_general(ones, x * x, (((1,), (1,)), ((), ())),
                              preferred_element_type=jnp.float32)
        g = jnp.min(cn - 2.0 * cxT, axis=0, keepdims=True)
        d2_ref[pl.ds(i, 1), :] = jnp.maximum(g + xnT, 0.0)

    d2a = d2_ref[...]
    bits = lax.bitcast_convert_type(d2a, jnp.int32)
    rbits = jnp.where(real, bits, jnp.int32(-1))
    fbits = jnp.where(real, jnp.int32(0x7FFFFFFF), bits)

    bmin = lax.bitcast_convert_type(jnp.min(d2a), jnp.int32)
    bmax = lax.bitcast_convert_type(jnp.max(d2a), jnp.int32) + 1

    def it(_, carry):
        lo_r, hi_r, lo_f, hi_f = carry
        mid_r = lo_r + (hi_r - lo_r) // 2
        mid_f = lo_f + (hi_f - lo_f) // 2
        contrib = ((rbits >= mid_r).astype(jnp.int32)
                   + ((fbits < mid_f).astype(jnp.int32) << 15))
        s = jnp.sum(contrib)
        cnt_r = s & 32767
        cnt_f = s >> 15
        ge = cnt_r >= k_real
        lo_r = jnp.where(ge, mid_r, lo_r)
        hi_r = jnp.where(ge, hi_r, mid_r)
        lt = cnt_f < k_forged
        lo_f = jnp.where(lt, mid_f, lo_f)
        hi_f = jnp.where(lt, hi_f, mid_f)
        return lo_r, hi_r, lo_f, hi_f

    lo_r, _, lo_f, _ = lax.fori_loop(0, 16, it, (bmin, bmax, bmin, bmax))

    gt = rbits > lo_r
    ltm = fbits < lo_f
    cboth = ((gt.astype(jnp.int32)) + (ltm.astype(jnp.int32) << 15))
    sboth = jnp.sum(cboth)
    cnt_gt = (sboth & 32767).astype(jnp.float32)
    cnt_lt = (sboth >> 15).astype(jnp.float32)

    v_r = lax.bitcast_convert_type(lo_r, jnp.float32)
    sum_gt = jnp.sum(jnp.where(gt, d2a, 0.0))
    top_sum = sum_gt + (k_real_f - cnt_gt) * v_r
    real_loss = top_sum / (2.0 * (k_real_f + _EPS))
    real_loss = jnp.where(num_real > 0, real_loss, 0.0)

    v_f = lax.bitcast_convert_type(lo_f, jnp.float32)
    sum_lt = jnp.sum(jnp.where(ltm, d2a, 0.0))
    bot_sum = sum_lt + (k_forged_f - cnt_lt) * v_f
    avg_forged = bot_sum / (2.0 * (k_forged_f + _EPS))
    forged_term = jnp.where(num_forged > 0,
                            jnp.minimum(avg_forged, _GAMMA2), 0.0)

    cc = lax.dot_general(c, c, (((1,), (1,)), ((), ())),
                         preferred_element_type=jnp.float32)
    cn2 = jnp.sum(c * c, axis=1)
    d2m = jnp.maximum(cn2[:, None] + cn2[None, :] - 2.0 * cc, 0.0)
    ii = lax.broadcasted_iota(jnp.int32, (_NC, _NC), 0)
    jj = lax.broadcasted_iota(jnp.int32, (_NC, _NC), 1)
    upper = jj > ii
    dist = jnp.sqrt(d2m + _EPS)
    hinge = jnp.maximum(_CENTER_MARGIN - dist, 0.0)
    num_pairs = _NC * (_NC - 1) // 2
    repulsion = _LAMBDA_CENTER * (
        jnp.sum(jnp.where(upper, hinge, 0.0)) / (num_pairs + _EPS))

    out_ref[0, 0] = real_loss - forged_term + repulsion


def kernel(cls_global, labels, centers):
    labels2d = labels.reshape(_NCH, _CH)
    out = pl.pallas_call(
        _body,
        in_specs=[
            pl.BlockSpec(memory_space=pltpu.VMEM),
            pl.BlockSpec(memory_space=pltpu.VMEM),
            pl.BlockSpec(memory_space=pl.ANY),
        ],
        out_specs=pl.BlockSpec(memory_space=pltpu.SMEM),
        out_shape=jax.ShapeDtypeStruct((1, 1), jnp.float32),
        scratch_shapes=[
            pltpu.VMEM((_NBUF, _CH, _D), jnp.float32),
            pltpu.VMEM((_NCH, _CH), jnp.float32),
            pltpu.SemaphoreType.DMA((_NBUF,)),
        ],
    )(labels2d, centers, cls_global)
    return out[0, 0]
